# baseline (speedup 1.0000x reference)
import functools

import jax
import jax.numpy as jnp
from jax import lax
from jax.experimental import pallas as pl
from jax.experimental.pallas import tpu as pltpu
from jax.experimental.pallas import tpu_sc as plsc

_B, _C, _HW = 32, 16, 128 * 128
_K = 500
_KP = 512
_NCHUNK = _KP // 16
_NROW = _C * _KP // 128


def _sc_body(out_hbm, idx_hbm, mask_hbm, tgt_hbm, part_hbm,
             idx_v, mask_v, tgt_v, ck_v, cvi_v, idxg_v, pred_v, out_v, sem0, sem1):
    b = lax.axis_index("s") * 2 + lax.axis_index("c")

    pltpu.sync_copy(idx_hbm.at[pl.ds(b * _KP, _KP)], idx_v)
    pltpu.sync_copy(mask_hbm.at[pl.ds(b * _KP, _KP)], mask_v)
    pltpu.sync_copy(tgt_hbm.at[b], tgt_v)

    lane = lax.iota(jnp.int32, 16)

    def compact(j, cnt):
        mb = mask_v[pl.ds(j * 16, 16)] > 0
        plsc.store_compressed(ck_v.at[pl.ds(cnt, 16)], lane + j * 16, mask=mb)
        plsc.store_compressed(cvi_v.at[pl.ds(cnt, 16)], idx_v[pl.ds(j * 16, 16)], mask=mb)
        pc = plsc.all_reduce_population_count(mb)
        return cnt + lax.reduce_max(pc, (0,))

    cnt = lax.fori_loop(0, _NCHUNK, compact, jnp.int32(0))
    ck_v[pl.ds(cnt, 16)] = jnp.full((16,), _KP - 1, jnp.int32)
    cvi_v[pl.ds(cnt, 16)] = jnp.zeros((16,), jnp.int32)
    cnt_pad = ((cnt + 15) // 16) * 16
    nch = cnt_pad // 16

    for c in range(_C):
        base = (b * _C + c) * _HW

        def bld(jj, _, c=c, base=base):
            idxg_v[pl.ds(c * cnt_pad + jj * 16, 16)] = (
                cvi_v[pl.ds(jj * 16, 16)] + base)
            return 0

        lax.fori_loop(0, nch, bld, 0)

    sems = (sem0, sem1)
    half_words = 8 * cnt_pad

    def issue(d, _, h=0):
        base = h * half_words
        pltpu.async_copy(out_hbm.at[idxg_v.at[pl.ds(base + d * 128, 128)]],
                         pred_v.at[pl.ds(base + d * 128, 128)], sems[h])
        return 0

    def drain(d, _, h=0):
        pltpu.make_async_copy(out_hbm.at[pl.ds(0, 128)],
                              pred_v.at[pl.ds(0, 128)], sems[h]).wait()
        return 0

    lax.fori_loop(0, nch, functools.partial(issue, h=0), 0)
    lax.fori_loop(0, nch, functools.partial(issue, h=1), 0)

    acc = jnp.zeros((16,), jnp.float32)
    for h in range(2):
        lax.fori_loop(0, nch, functools.partial(drain, h=h), 0)
        for c in range(h * 8, (h + 1) * 8):
            def chunk(jj, a, c=c):
                p = pred_v[pl.ds(c * cnt_pad + jj * 16, 16)]
                ck = ck_v[pl.ds(jj * 16, 16)]
                m = plsc.load_gather(mask_v, [ck]).astype(jnp.float32)
                t = plsc.load_gather(
                    tgt_v, [jnp.minimum((ck << 4) + c, _K * _C - 1)])
                return a + m * jnp.abs(p - t)

            acc = lax.fori_loop(0, nch, chunk, acc)

    msum = lax.fori_loop(
        0, _NCHUNK,
        lambda j, a: a + mask_v[pl.ds(j * 16, 16)],
        jnp.zeros((16,), jnp.int32)).astype(jnp.float32)

    out_v[pl.ds(0, 16)] = acc
    out_v[pl.ds(16, 16)] = msum
    pltpu.sync_copy(out_v, part_hbm.at[b])


_launch = functools.partial(
    pl.kernel,
    mesh=plsc.VectorSubcoreMesh(core_axis_name="c", subcore_axis_name="s"),
    out_type=jax.ShapeDtypeStruct((_B, 32), jnp.float32),
    scratch_types=[
        pltpu.VMEM((_KP,), jnp.int32),
        pltpu.VMEM((_KP,), jnp.int32),
        pltpu.VMEM((_K * _C,), jnp.float32),
        pltpu.VMEM((_KP + 16,), jnp.int32),
        pltpu.VMEM((_KP + 16,), jnp.int32),
        pltpu.VMEM((_C * _KP,), jnp.int32),
        pltpu.VMEM((_C * _KP,), jnp.float32),
        pltpu.VMEM((32,), jnp.float32),
        pltpu.SemaphoreType.DMA,
        pltpu.SemaphoreType.DMA,
    ],
    compiler_params=pltpu.CompilerParams(needs_layout_passes=False),
)(_sc_body)


@jax.jit
def kernel(output, mask, index, target):
    pad = _KP - _K
    out_flat = output.reshape(-1)
    idx_p = jnp.pad(index.astype(jnp.int32), ((0, 0), (0, pad))).reshape(-1)
    mask_p = jnp.pad(mask.astype(jnp.int32), ((0, 0), (0, pad))).reshape(-1)
    tgt_p = target.reshape(_B, _K * _C)
    parts = _launch(out_flat, idx_p, mask_p, tgt_p)
    s = jnp.sum(parts[:, :16])
    m = jnp.sum(parts[:, 16:])
    return s / (_C * m + 0.0001)


# speedup vs baseline: 1.0975x; 1.0975x over previous
import functools

import jax
import jax.numpy as jnp
from jax import lax
from jax.experimental import pallas as pl
from jax.experimental.pallas import tpu as pltpu
from jax.experimental.pallas import tpu_sc as plsc

_B, _C, _HW = 32, 16, 128 * 128
_K = 500
_KP = 512
_NCHUNK = _KP // 16
_NROW = _C * _KP // 128


def _sc_body(out_hbm, idx_hbm, mask_hbm, tgt_hbm, part_hbm,
             idx_v, mask_v, tgt_v, ck_v, cvi_v, idxg_v, pred_v, out_v, sem0, sem1):
    b = lax.axis_index("s") * 2 + lax.axis_index("c")

    pltpu.sync_copy(idx_hbm.at[pl.ds(b * _KP, _KP)], idx_v)
    pltpu.sync_copy(mask_hbm.at[pl.ds(b * _KP, _KP)], mask_v)
    pltpu.sync_copy(tgt_hbm.at[b], tgt_v)

    lane = lax.iota(jnp.int32, 16)

    def compact(j, cnt):
        mb = mask_v[pl.ds(j * 16, 16)] > 0.5
        plsc.store_compressed(ck_v.at[pl.ds(cnt, 16)], lane + j * 16, mask=mb)
        plsc.store_compressed(cvi_v.at[pl.ds(cnt, 16)], idx_v[pl.ds(j * 16, 16)], mask=mb)
        pc = plsc.all_reduce_population_count(mb)
        return cnt + lax.reduce_max(pc, (0,))

    cnt = lax.fori_loop(0, _NCHUNK, compact, jnp.int32(0))
    ck_v[pl.ds(cnt, 16)] = jnp.full((16,), _KP - 1, jnp.int32)
    cvi_v[pl.ds(cnt, 16)] = jnp.zeros((16,), jnp.int32)
    cnt_pad = ((cnt + 15) // 16) * 16
    nch = cnt_pad // 16

    for c in range(_C):
        base = (b * _C + c) * _HW

        def bld(jj, _, c=c, base=base):
            idxg_v[pl.ds(c * cnt_pad + jj * 16, 16)] = (
                cvi_v[pl.ds(jj * 16, 16)] + base)
            return 0

        lax.fori_loop(0, nch, bld, 0)

    sems = (sem0, sem1)
    half_words = 8 * cnt_pad

    def issue(d, _, h=0):
        base = h * half_words
        pltpu.async_copy(out_hbm.at[idxg_v.at[pl.ds(base + d * 128, 128)]],
                         pred_v.at[pl.ds(base + d * 128, 128)], sems[h])
        return 0

    def drain(d, _, h=0):
        pltpu.make_async_copy(out_hbm.at[pl.ds(0, 128)],
                              pred_v.at[pl.ds(0, 128)], sems[h]).wait()
        return 0

    lax.fori_loop(0, nch, functools.partial(issue, h=0), 0)
    lax.fori_loop(0, nch, functools.partial(issue, h=1), 0)

    acc = jnp.zeros((16,), jnp.float32)
    for h in range(2):
        lax.fori_loop(0, nch, functools.partial(drain, h=h), 0)
        for c in range(h * 8, (h + 1) * 8):
            def chunk(jj, a, c=c):
                p = pred_v[pl.ds(c * cnt_pad + jj * 16, 16)]
                ck = ck_v[pl.ds(jj * 16, 16)]
                m = plsc.load_gather(mask_v, [ck])
                t = plsc.load_gather(tgt_v, [c * 4 + (ck >> 7), ck & 127])
                return a + jnp.abs(m * p - t)

            acc = lax.fori_loop(0, nch, chunk, acc)

    msum = lax.fori_loop(
        0, _NCHUNK,
        lambda j, a: a + mask_v[pl.ds(j * 16, 16)],
        jnp.zeros((16,), jnp.float32))

    out_v[pl.ds(0, 16)] = acc
    out_v[pl.ds(16, 16)] = msum
    pltpu.sync_copy(out_v, part_hbm.at[b])


_launch = functools.partial(
    pl.kernel,
    mesh=plsc.VectorSubcoreMesh(core_axis_name="c", subcore_axis_name="s"),
    out_type=jax.ShapeDtypeStruct((_B, 32), jnp.float32),
    scratch_types=[
        pltpu.VMEM((_KP,), jnp.int32),
        pltpu.VMEM((_KP,), jnp.float32),
        pltpu.VMEM((_NROW, 128), jnp.float32),
        pltpu.VMEM((_KP + 16,), jnp.int32),
        pltpu.VMEM((_KP + 16,), jnp.int32),
        pltpu.VMEM((_C * _KP,), jnp.int32),
        pltpu.VMEM((_C * _KP,), jnp.float32),
        pltpu.VMEM((32,), jnp.float32),
        pltpu.SemaphoreType.DMA,
        pltpu.SemaphoreType.DMA,
    ],
    compiler_params=pltpu.CompilerParams(needs_layout_passes=False),
)(_sc_body)


@jax.jit
def kernel(output, mask, index, target):
    pad = _KP - _K
    out_flat = output.reshape(-1)
    idx_p = jnp.pad(index.astype(jnp.int32), ((0, 0), (0, pad))).reshape(-1)
    mask_f = mask.astype(jnp.float32)
    mask_p = jnp.pad(mask_f, ((0, 0), (0, pad))).reshape(-1)
    tgt_t = jnp.transpose(target * mask_f[:, :, None], (0, 2, 1))
    tgt_p = jnp.pad(tgt_t, ((0, 0), (0, 0), (0, pad))).reshape(_B, _NROW, 128)
    parts = _launch(out_flat, idx_p, mask_p, tgt_p)
    s = jnp.sum(parts[:, :16])
    m = jnp.sum(parts[:, 16:])
    return s / (_C * m + 0.0001)


# speedup vs baseline: 1.1203x; 1.0208x over previous
import functools

import jax
import jax.numpy as jnp
from jax import lax
from jax.experimental import pallas as pl
from jax.experimental.pallas import tpu as pltpu
from jax.experimental.pallas import tpu_sc as plsc

_B, _C, _HW = 32, 16, 128 * 128
_K = 500
_KP = 512
_NCHUNK = _KP // 16
_NROW = _C * _KP // 128


def _sc_body(out_hbm, idx_hbm, mask_hbm, tgt_hbm, part_hbm,
             idx_v, mask_v, tgt_v, ck_v, cvi_v, idxg_v, pred_v, out_v, sem0, sem1):
    b = lax.axis_index("s") * 2 + lax.axis_index("c")

    pltpu.sync_copy(idx_hbm.at[pl.ds(b * _KP, _KP)], idx_v)
    pltpu.sync_copy(mask_hbm.at[pl.ds(b * _KP, _KP)], mask_v)
    pltpu.sync_copy(tgt_hbm.at[b], tgt_v)

    lane = lax.iota(jnp.int32, 16)

    def compact(j, cnt):
        mb = mask_v[pl.ds(j * 16, 16)] > 0.5
        plsc.store_compressed(ck_v.at[pl.ds(cnt, 16)], lane + j * 16, mask=mb)
        plsc.store_compressed(cvi_v.at[pl.ds(cnt, 16)], idx_v[pl.ds(j * 16, 16)], mask=mb)
        pc = plsc.all_reduce_population_count(mb)
        return cnt + lax.reduce_max(pc, (0,))

    cnt = lax.fori_loop(0, _NCHUNK, compact, jnp.int32(0))
    ck_v[pl.ds(cnt, 16)] = jnp.full((16,), _KP - 1, jnp.int32)
    cvi_v[pl.ds(cnt, 16)] = jnp.zeros((16,), jnp.int32)
    cnt_pad = ((cnt + 15) // 16) * 16
    nch = cnt_pad // 16

    sems = (sem0, sem1)
    half_words = 8 * cnt_pad

    def issue(d, _, h=0):
        base = h * half_words
        pltpu.async_copy(out_hbm.at[idxg_v.at[pl.ds(base + d * 128, 128)]],
                         pred_v.at[pl.ds(base + d * 128, 128)], sems[h])
        return 0

    def drain(d, _, h=0):
        pltpu.make_async_copy(out_hbm.at[pl.ds(0, 128)],
                              pred_v.at[pl.ds(0, 128)], sems[h]).wait()
        return 0

    for h in range(2):
        for c in range(h * 8, (h + 1) * 8):
            base = (b * _C + c) * _HW

            def bld(jj, _, c=c, base=base):
                idxg_v[pl.ds(c * cnt_pad + jj * 16, 16)] = (
                    cvi_v[pl.ds(jj * 16, 16)] + base)
                return 0

            lax.fori_loop(0, nch, bld, 0)
        lax.fori_loop(0, nch, functools.partial(issue, h=h), 0)

    msum = lax.fori_loop(
        0, _NCHUNK,
        lambda j, a: a + mask_v[pl.ds(j * 16, 16)],
        jnp.zeros((16,), jnp.float32))

    acc = jnp.zeros((16,), jnp.float32)
    for h in range(2):
        lax.fori_loop(0, nch, functools.partial(drain, h=h), 0)
        for c in range(h * 8, (h + 1) * 8):
            def chunk(jj, a, c=c):
                p = pred_v[pl.ds(c * cnt_pad + jj * 16, 16)]
                ck = ck_v[pl.ds(jj * 16, 16)]
                m = plsc.load_gather(mask_v, [ck])
                t = plsc.load_gather(tgt_v, [c * 4 + (ck >> 7), ck & 127])
                return a + jnp.abs(m * p - t)

            acc = lax.fori_loop(0, nch, chunk, acc)

    out_v[pl.ds(0, 16)] = acc
    out_v[pl.ds(16, 16)] = msum
    pltpu.sync_copy(out_v, part_hbm.at[b])


_launch = functools.partial(
    pl.kernel,
    mesh=plsc.VectorSubcoreMesh(core_axis_name="c", subcore_axis_name="s"),
    out_type=jax.ShapeDtypeStruct((_B, 32), jnp.float32),
    scratch_types=[
        pltpu.VMEM((_KP,), jnp.int32),
        pltpu.VMEM((_KP,), jnp.float32),
        pltpu.VMEM((_NROW, 128), jnp.float32),
        pltpu.VMEM((_KP + 16,), jnp.int32),
        pltpu.VMEM((_KP + 16,), jnp.int32),
        pltpu.VMEM((_C * _KP,), jnp.int32),
        pltpu.VMEM((_C * _KP,), jnp.float32),
        pltpu.VMEM((32,), jnp.float32),
        pltpu.SemaphoreType.DMA,
        pltpu.SemaphoreType.DMA,
    ],
    compiler_params=pltpu.CompilerParams(needs_layout_passes=False),
)(_sc_body)


@jax.jit
def kernel(output, mask, index, target):
    pad = _KP - _K
    out_flat = output.reshape(-1)
    idx_p = jnp.pad(index.astype(jnp.int32), ((0, 0), (0, pad))).reshape(-1)
    mask_f = mask.astype(jnp.float32)
    mask_p = jnp.pad(mask_f, ((0, 0), (0, pad))).reshape(-1)
    tgt_t = jnp.transpose(target * mask_f[:, :, None], (0, 2, 1))
    tgt_p = jnp.pad(tgt_t, ((0, 0), (0, 0), (0, pad))).reshape(_B, _NROW, 128)
    parts = _launch(out_flat, idx_p, mask_p, tgt_p)
    s = jnp.sum(parts[:, :16])
    m = jnp.sum(parts[:, 16:])
    return s / (_C * m + 0.0001)
